# Initial kernel scaffold; baseline (speedup 1.0000x reference)
import jax
import jax.numpy as jnp
from jax.experimental import pallas as pl


def kernel(x, edge_index, feature_importance, W_mean, b_mean, W_ego, b_ego, W_nbr, b_nbr, gate_w, gate_b, bn_gamma, bn_beta, W_gcn, b_gcn):
    raise NotImplementedError("write your pallas kernel here")


# speedup vs baseline: 10.3781x; 10.3781x over previous
import functools

import jax
import jax.numpy as jnp
from jax import lax
from jax.experimental import pallas as pl
from jax.experimental.pallas import tpu as pltpu
from jax.experimental.pallas import tpu_sc as plsc

N = 10000
E = 320000
F = 128
HID = 128
OUT = 2
EGO = HID // 2

NC = 2
NS = 16
NW = NC * NS
CH = 128
CPT = 80
E_PAD = NW * CPT * CH
ROWS_PER_SUB = 632
N_PAD = NS * ROWS_PER_SUB

_mesh = plsc.VectorSubcoreMesh(core_axis_name="c", subcore_axis_name="s")


def _ids():
    cid = lax.axis_index("c")
    sid = lax.axis_index("s")
    wid = sid * NC + cid
    return cid, sid, wid


FH = F // 2


@functools.partial(
    pl.kernel,
    out_type=(
        jax.ShapeDtypeStruct((NC, N_PAD, FH), jnp.float32),
        jax.ShapeDtypeStruct((NC, N_PAD, FH), jnp.float32),
        jax.ShapeDtypeStruct((NC, N_PAD, 8), jnp.float32),
        jax.ShapeDtypeStruct((NC, N_PAD, 8), jnp.float32),
    ),
    mesh=_mesh,
    scratch_types=[
        pltpu.VMEM((CPT, CH), jnp.int32),
        pltpu.VMEM((CPT, CH), jnp.int32),
        pltpu.VMEM((CH, FH), jnp.float32),
        pltpu.VMEM((CH, 8), jnp.float32),
        pltpu.VMEM_SHARED((N_PAD, FH), jnp.float32),
        pltpu.VMEM_SHARED((N_PAD, 8), jnp.float32),
    ],
    compiler_params=pltpu.CompilerParams(use_tc_tiling_on_sc=False),
)
def _agg_kernel(xlo, xhi, src3d, dst3d, zeros_f, zeros_d, ones_hbm,
                acclo_out, acchi_out, degs_out, degd_out,
                src_b, dst_b, rows_b, ones_b, acc_sh, deg_sh):
    cid, sid, wid = _ids()
    row0 = sid * ROWS_PER_SUB
    sl = pl.ds(row0, ROWS_PER_SUB)
    pltpu.sync_copy(zeros_f, acc_sh.at[sl])
    pltpu.sync_copy(zeros_d, deg_sh.at[sl])
    pltpu.sync_copy(src3d.at[wid], src_b)
    pltpu.sync_copy(dst3d.at[wid], dst_b)
    pltpu.sync_copy(ones_hbm, ones_b)
    plsc.subcore_barrier()

    def body1(k, carry):
        src_row = src_b.at[k]
        pltpu.sync_copy(xlo.at[dst_b.at[k]], rows_b)
        pltpu.sync_copy(rows_b, acc_sh.at[src_row], add=True)
        pltpu.sync_copy(ones_b, deg_sh.at[src_row], add=True)
        return carry

    lax.fori_loop(0, CPT, body1, 0)
    plsc.subcore_barrier()
    pltpu.sync_copy(acc_sh.at[sl], acclo_out.at[cid, sl])
    pltpu.sync_copy(deg_sh.at[sl], degs_out.at[cid, sl])
    pltpu.sync_copy(zeros_f, acc_sh.at[sl])
    pltpu.sync_copy(zeros_d, deg_sh.at[sl])
    plsc.subcore_barrier()

    def body2(k, carry):
        dst_row = dst_b.at[k]
        pltpu.sync_copy(xhi.at[dst_row], rows_b)
        pltpu.sync_copy(rows_b, acc_sh.at[src_b.at[k]], add=True)
        pltpu.sync_copy(ones_b, deg_sh.at[dst_row], add=True)
        return carry

    lax.fori_loop(0, CPT, body2, 0)
    plsc.subcore_barrier()
    pltpu.sync_copy(acc_sh.at[sl], acchi_out.at[cid, sl])
    pltpu.sync_copy(deg_sh.at[sl], degd_out.at[cid, sl])


@functools.partial(
    pl.kernel,
    out_type=jax.ShapeDtypeStruct((NC, N_PAD, 16), jnp.float32),
    mesh=_mesh,
    scratch_types=[
        pltpu.VMEM((CPT, CH), jnp.int32),
        pltpu.VMEM((CPT, CH), jnp.int32),
        pltpu.VMEM((CH, 16), jnp.float32),
        pltpu.VMEM_SHARED((N_PAD, 16), jnp.float32),
    ],
    compiler_params=pltpu.CompilerParams(use_tc_tiling_on_sc=False),
)
def _gcn_agg_kernel(z_hbm, src3d, dst3d, zeros_d, agg_out,
                    src_b, dst_b, rows_b, agg_sh):
    cid, sid, wid = _ids()
    row0 = sid * ROWS_PER_SUB
    pltpu.sync_copy(zeros_d, agg_sh.at[pl.ds(row0, ROWS_PER_SUB)])
    pltpu.sync_copy(src3d.at[wid], src_b)
    pltpu.sync_copy(dst3d.at[wid], dst_b)
    plsc.subcore_barrier()

    def body(k, carry):
        pltpu.sync_copy(z_hbm.at[src_b.at[k]], rows_b)
        pltpu.sync_copy(rows_b, agg_sh.at[dst_b.at[k]], add=True)
        return carry

    lax.fori_loop(0, CPT, body, 0)
    plsc.subcore_barrier()
    sl = pl.ds(row0, ROWS_PER_SUB)
    pltpu.sync_copy(agg_sh.at[sl], agg_out.at[cid, sl])


def _dense_body(x_ref, acclo_ref, acchi_ref, degsp_ref, degdp_ref, fi_ref,
                wm_ref, bm_ref, we_ref, be_ref, wn_ref, bnb_ref,
                gw_ref, gb_ref, gam_ref, bet_ref, wg_ref,
                z_ref, aux_ref):
    sig_fi = jax.nn.sigmoid(fi_ref[...])
    xs = x_ref[...] * sig_fi
    nbr = jnp.concatenate(
        [acclo_ref[0] + acclo_ref[1], acchi_ref[0] + acchi_ref[1]], axis=1) * sig_fi
    deg = degsp_ref[0, :, 0:1] + degsp_ref[1, :, 0:1]
    mean = nbr / jnp.maximum(deg, 1.0)
    xn = xs / jnp.maximum(jnp.sqrt(jnp.sum(xs * xs, axis=1, keepdims=True)), 1e-12)
    mn = mean / jnp.maximum(jnp.sqrt(jnp.sum(mean * mean, axis=1, keepdims=True)), 1e-12)
    sim = jnp.sum(xn * mn, axis=1, keepdims=True)
    sim = jnp.where(deg > 0, sim, 1.0)
    delta = jax.nn.sigmoid(deg * (1.0 - sim) / 10.0 - 0.5)

    dot = lambda a, w: lax.dot_general(a, w, (((1,), (1,)), ((), ())),
                                       preferred_element_type=jnp.float32)
    h_mean = dot((xs + mean) * 0.5, wm_ref[...]) + bm_ref[...]
    h_ego = dot(xs, we_ref[...]) + be_ref[...]
    h_nbr = dot(mean, wn_ref[...]) + bnb_ref[...]
    h_cat = jnp.concatenate([h_ego, h_nbr], axis=1)
    gate = jax.nn.sigmoid(gw_ref[0, 0] * delta + gb_ref[0, 0])
    h = (1.0 - gate) * h_mean + gate * h_cat
    h = h * (1.0 / jnp.sqrt(1.0 + 1e-5)) * gam_ref[...] + bet_ref[...]
    h = jnp.maximum(h, 0.0)
    hw = dot(h, wg_ref[...])

    degd = degdp_ref[0, :, 0:1] + degdp_ref[1, :, 0:1]
    dinv = 1.0 / jnp.sqrt(degd + 1.0)
    z = dinv * hw
    r = z.shape[0]
    z_ref[...] = jnp.concatenate([z, jnp.zeros((r, 16 - OUT), jnp.float32)], axis=1)
    aux_ref[...] = jnp.concatenate(
        [dinv, jnp.zeros((r, 1), jnp.float32), dinv * z,
         jnp.zeros((r, 12), jnp.float32)], axis=1)


def _final_body(aggp_ref, aux_ref, bg_ref, out_ref):
    agg = aggp_ref[0] + aggp_ref[1]
    out_ref[...] = aux_ref[:, 0:1] * agg[:, 0:OUT] + aux_ref[:, 2:2 + OUT] + bg_ref[...]


def kernel(x, edge_index, feature_importance, W_mean, b_mean, W_ego, b_ego,
           W_nbr, b_nbr, gate_w, gate_b, bn_gamma, bn_beta, W_gcn, b_gcn):
    dummy = jnp.full((E_PAD - E,), N, jnp.int32)
    src3d = jnp.concatenate([edge_index[0], dummy]).reshape(NW, CPT, CH)
    dst3d = jnp.concatenate([edge_index[1], dummy]).reshape(NW, CPT, CH)
    xpad = jnp.concatenate([x, jnp.zeros((N_PAD - N, F), jnp.float32)])
    zeros_f = jnp.zeros((ROWS_PER_SUB, FH), jnp.float32)
    zeros_8 = jnp.zeros((ROWS_PER_SUB, 8), jnp.float32)
    zeros_d = jnp.zeros((ROWS_PER_SUB, 16), jnp.float32)
    ones8 = jnp.ones((CH, 8), jnp.float32)

    acclo, acchi, degsp, degdp = _agg_kernel(
        xpad[:, :FH], xpad[:, FH:], src3d, dst3d, zeros_f, zeros_8, ones8)

    R = 1000
    grid = N // R
    full = lambda shape: pl.BlockSpec(shape, lambda i: (0,) * len(shape))
    zpad, aux = pl.pallas_call(
        _dense_body,
        grid=(grid,),
        in_specs=[
            pl.BlockSpec((R, F), lambda i: (i, 0)),
            pl.BlockSpec((NC, R, FH), lambda i: (0, i, 0)),
            pl.BlockSpec((NC, R, FH), lambda i: (0, i, 0)),
            pl.BlockSpec((NC, R, 8), lambda i: (0, i, 0)),
            pl.BlockSpec((NC, R, 8), lambda i: (0, i, 0)),
            full((1, F)),
            full((HID, F)), full((1, HID)),
            full((EGO, F)), full((1, EGO)),
            full((HID - EGO, F)), full((1, HID - EGO)),
            full((1, 1)), full((1, 1)),
            full((1, HID)), full((1, HID)),
            full((OUT, HID)),
        ],
        out_specs=[
            pl.BlockSpec((R, 16), lambda i: (i, 0)),
            pl.BlockSpec((R, 16), lambda i: (i, 0)),
        ],
        out_shape=[
            jax.ShapeDtypeStruct((N_PAD, 16), jnp.float32),
            jax.ShapeDtypeStruct((N, 16), jnp.float32),
        ],
    )(x, acclo, acchi, degsp, degdp,
      feature_importance.reshape(1, F),
      W_mean, b_mean.reshape(1, HID),
      W_ego, b_ego.reshape(1, EGO),
      W_nbr, b_nbr.reshape(1, HID - EGO),
      gate_w.reshape(1, 1), gate_b.reshape(1, 1),
      bn_gamma.reshape(1, HID), bn_beta.reshape(1, HID),
      W_gcn)

    aggp = _gcn_agg_kernel(zpad, src3d, dst3d, zeros_d)

    out = pl.pallas_call(
        _final_body,
        grid=(grid,),
        in_specs=[
            pl.BlockSpec((NC, R, 16), lambda i: (0, i, 0)),
            pl.BlockSpec((R, 16), lambda i: (i, 0)),
            full((1, OUT)),
        ],
        out_specs=pl.BlockSpec((R, OUT), lambda i: (i, 0)),
        out_shape=jax.ShapeDtypeStruct((N, OUT), jnp.float32),
    )(aggp, aux, b_gcn.reshape(1, OUT))
    return out


# speedup vs baseline: 17.6797x; 1.7036x over previous
import functools

import jax
import jax.numpy as jnp
from jax import lax
from jax.experimental import pallas as pl
from jax.experimental.pallas import tpu as pltpu
from jax.experimental.pallas import tpu_sc as plsc

N = 10000
E = 320000
F = 128
HID = 128
OUT = 2
EGO = HID // 2

NC = 2
NS = 16
NW = NC * NS
CH = 128
CPT = 80
E_PAD = NW * CPT * CH
ROWS_PER_SUB = 632
N_PAD = NS * ROWS_PER_SUB

_mesh = plsc.VectorSubcoreMesh(core_axis_name="c", subcore_axis_name="s")


def _ids():
    cid = lax.axis_index("c")
    sid = lax.axis_index("s")
    wid = sid * NC + cid
    return cid, sid, wid


FH = F // 2


@functools.partial(
    pl.kernel,
    out_type=(
        jax.ShapeDtypeStruct((NC, N_PAD, FH), jnp.float32),
        jax.ShapeDtypeStruct((NC, N_PAD, FH), jnp.float32),
        jax.ShapeDtypeStruct((NC, N_PAD, 8), jnp.float32),
        jax.ShapeDtypeStruct((NC, N_PAD, 8), jnp.float32),
    ),
    mesh=_mesh,
    scratch_types=[
        pltpu.VMEM((CPT, CH), jnp.int32),
        pltpu.VMEM((CPT, CH), jnp.int32),
        pltpu.VMEM((CH, FH), jnp.float32),
        pltpu.VMEM((CH, 8), jnp.float32),
        pltpu.VMEM_SHARED((N_PAD, FH), jnp.float32),
        pltpu.VMEM_SHARED((N_PAD, 8), jnp.float32),
    ],
    compiler_params=pltpu.CompilerParams(use_tc_tiling_on_sc=False),
)
def _agg_kernel(xlo, xhi, src3d, dst3d, zeros_f, zeros_d, ones_hbm,
                acclo_out, acchi_out, degs_out, degd_out,
                src_b, dst_b, rows_b, ones_b, acc_sh, deg_sh):
    cid, sid, wid = _ids()
    row0 = sid * ROWS_PER_SUB
    sl = pl.ds(row0, ROWS_PER_SUB)
    pltpu.sync_copy(zeros_f, acc_sh.at[sl])
    pltpu.sync_copy(zeros_d, deg_sh.at[sl])
    pltpu.sync_copy(src3d.at[wid], src_b)
    pltpu.sync_copy(dst3d.at[wid], dst_b)
    pltpu.sync_copy(ones_hbm, ones_b)
    plsc.subcore_barrier()

    def body1(k, carry):
        src_row = src_b.at[k]
        pltpu.sync_copy(xlo.at[dst_b.at[k]], rows_b)
        pltpu.sync_copy(rows_b, acc_sh.at[src_row], add=True)
        pltpu.sync_copy(ones_b, deg_sh.at[src_row], add=True)
        return carry

    lax.fori_loop(0, CPT, body1, 0)
    plsc.subcore_barrier()
    pltpu.sync_copy(acc_sh.at[sl], acclo_out.at[cid, sl])
    pltpu.sync_copy(deg_sh.at[sl], degs_out.at[cid, sl])
    pltpu.sync_copy(zeros_f, acc_sh.at[sl])
    pltpu.sync_copy(zeros_d, deg_sh.at[sl])
    plsc.subcore_barrier()

    def body2(k, carry):
        dst_row = dst_b.at[k]
        pltpu.sync_copy(xhi.at[dst_row], rows_b)
        pltpu.sync_copy(rows_b, acc_sh.at[src_b.at[k]], add=True)
        pltpu.sync_copy(ones_b, deg_sh.at[dst_row], add=True)
        return carry

    lax.fori_loop(0, CPT, body2, 0)
    plsc.subcore_barrier()
    pltpu.sync_copy(acc_sh.at[sl], acchi_out.at[cid, sl])
    pltpu.sync_copy(deg_sh.at[sl], degd_out.at[cid, sl])


@functools.partial(
    pl.kernel,
    out_type=jax.ShapeDtypeStruct((NC, N_PAD, 16), jnp.float32),
    mesh=_mesh,
    scratch_types=[
        pltpu.VMEM((CPT, CH), jnp.int32),
        pltpu.VMEM((CPT, CH), jnp.int32),
        pltpu.VMEM((CH, 16), jnp.float32),
        pltpu.VMEM_SHARED((N_PAD, 16), jnp.float32),
    ],
    compiler_params=pltpu.CompilerParams(use_tc_tiling_on_sc=False),
)
def _gcn_agg_kernel(z_hbm, src3d, dst3d, zeros_d, agg_out,
                    src_b, dst_b, rows_b, agg_sh):
    cid, sid, wid = _ids()
    row0 = sid * ROWS_PER_SUB
    pltpu.sync_copy(zeros_d, agg_sh.at[pl.ds(row0, ROWS_PER_SUB)])
    pltpu.sync_copy(src3d.at[wid], src_b)
    pltpu.sync_copy(dst3d.at[wid], dst_b)
    plsc.subcore_barrier()

    def body(k, carry):
        pltpu.sync_copy(z_hbm.at[src_b.at[k]], rows_b)
        pltpu.sync_copy(rows_b, agg_sh.at[dst_b.at[k]], add=True)
        return carry

    lax.fori_loop(0, CPT, body, 0)
    plsc.subcore_barrier()
    sl = pl.ds(row0, ROWS_PER_SUB)
    pltpu.sync_copy(agg_sh.at[sl], agg_out.at[cid, sl])


def _dense_body(x_ref, acclo_ref, acchi_ref, degsp_ref, degdp_ref, fi_ref,
                wm_ref, bm_ref, we_ref, be_ref, wn_ref, bnb_ref,
                gw_ref, gb_ref, gam_ref, bet_ref, wg_ref,
                z_ref, aux_ref):
    sig_fi = jax.nn.sigmoid(fi_ref[...])
    xs = x_ref[...] * sig_fi
    nbr = jnp.concatenate(
        [acclo_ref[0] + acclo_ref[1], acchi_ref[0] + acchi_ref[1]], axis=1) * sig_fi
    deg = degsp_ref[0, :, 0:1] + degsp_ref[1, :, 0:1]
    mean = nbr / jnp.maximum(deg, 1.0)
    xn = xs / jnp.maximum(jnp.sqrt(jnp.sum(xs * xs, axis=1, keepdims=True)), 1e-12)
    mn = mean / jnp.maximum(jnp.sqrt(jnp.sum(mean * mean, axis=1, keepdims=True)), 1e-12)
    sim = jnp.sum(xn * mn, axis=1, keepdims=True)
    sim = jnp.where(deg > 0, sim, 1.0)
    delta = jax.nn.sigmoid(deg * (1.0 - sim) / 10.0 - 0.5)

    dot = lambda a, w: lax.dot_general(a, w, (((1,), (1,)), ((), ())),
                                       preferred_element_type=jnp.float32)
    h_mean = dot((xs + mean) * 0.5, wm_ref[...]) + bm_ref[...]
    h_ego = dot(xs, we_ref[...]) + be_ref[...]
    h_nbr = dot(mean, wn_ref[...]) + bnb_ref[...]
    h_cat = jnp.concatenate([h_ego, h_nbr], axis=1)
    gate = jax.nn.sigmoid(gw_ref[0, 0] * delta + gb_ref[0, 0])
    h = (1.0 - gate) * h_mean + gate * h_cat
    h = h * (1.0 / jnp.sqrt(1.0 + 1e-5)) * gam_ref[...] + bet_ref[...]
    h = jnp.maximum(h, 0.0)
    hw = dot(h, wg_ref[...])

    degd = degdp_ref[0, :, 0:1] + degdp_ref[1, :, 0:1]
    dinv = 1.0 / jnp.sqrt(degd + 1.0)
    z = dinv * hw
    r = z.shape[0]
    z_ref[...] = jnp.concatenate([z, jnp.zeros((r, 16 - OUT), jnp.float32)], axis=1)
    aux_ref[...] = jnp.concatenate(
        [dinv, jnp.zeros((r, 1), jnp.float32), dinv * z,
         jnp.zeros((r, 12), jnp.float32)], axis=1)


def _final_body(aggp_ref, aux_ref, bg_ref, out_ref):
    agg = aggp_ref[0] + aggp_ref[1]
    out_ref[...] = aux_ref[:, 0:1] * agg[:, 0:OUT] + aux_ref[:, 2:2 + OUT] + bg_ref[...]


def kernel(x, edge_index, feature_importance, W_mean, b_mean, W_ego, b_ego,
           W_nbr, b_nbr, gate_w, gate_b, bn_gamma, bn_beta, W_gcn, b_gcn):
    dummy = N + (jnp.arange(E_PAD - E, dtype=jnp.int32) % (N_PAD - N))
    src3d = jnp.concatenate([edge_index[0], dummy]).reshape(NW, CPT, CH)
    dst3d = jnp.concatenate([edge_index[1], dummy]).reshape(NW, CPT, CH)
    xpad = jnp.concatenate([x, jnp.zeros((N_PAD - N, F), jnp.float32)])
    zeros_f = jnp.zeros((ROWS_PER_SUB, FH), jnp.float32)
    zeros_8 = jnp.zeros((ROWS_PER_SUB, 8), jnp.float32)
    zeros_d = jnp.zeros((ROWS_PER_SUB, 16), jnp.float32)
    ones8 = jnp.ones((CH, 8), jnp.float32)

    acclo, acchi, degsp, degdp = _agg_kernel(
        xpad[:, :FH], xpad[:, FH:], src3d, dst3d, zeros_f, zeros_8, ones8)

    R = 1000
    grid = N // R
    full = lambda shape: pl.BlockSpec(shape, lambda i: (0,) * len(shape))
    zpad, aux = pl.pallas_call(
        _dense_body,
        grid=(grid,),
        in_specs=[
            pl.BlockSpec((R, F), lambda i: (i, 0)),
            pl.BlockSpec((NC, R, FH), lambda i: (0, i, 0)),
            pl.BlockSpec((NC, R, FH), lambda i: (0, i, 0)),
            pl.BlockSpec((NC, R, 8), lambda i: (0, i, 0)),
            pl.BlockSpec((NC, R, 8), lambda i: (0, i, 0)),
            full((1, F)),
            full((HID, F)), full((1, HID)),
            full((EGO, F)), full((1, EGO)),
            full((HID - EGO, F)), full((1, HID - EGO)),
            full((1, 1)), full((1, 1)),
            full((1, HID)), full((1, HID)),
            full((OUT, HID)),
        ],
        out_specs=[
            pl.BlockSpec((R, 16), lambda i: (i, 0)),
            pl.BlockSpec((R, 16), lambda i: (i, 0)),
        ],
        out_shape=[
            jax.ShapeDtypeStruct((N_PAD, 16), jnp.float32),
            jax.ShapeDtypeStruct((N, 16), jnp.float32),
        ],
    )(x, acclo, acchi, degsp, degdp,
      feature_importance.reshape(1, F),
      W_mean, b_mean.reshape(1, HID),
      W_ego, b_ego.reshape(1, EGO),
      W_nbr, b_nbr.reshape(1, HID - EGO),
      gate_w.reshape(1, 1), gate_b.reshape(1, 1),
      bn_gamma.reshape(1, HID), bn_beta.reshape(1, HID),
      W_gcn)

    aggp = _gcn_agg_kernel(zpad, src3d, dst3d, zeros_d)

    out = pl.pallas_call(
        _final_body,
        grid=(grid,),
        in_specs=[
            pl.BlockSpec((NC, R, 16), lambda i: (0, i, 0)),
            pl.BlockSpec((R, 16), lambda i: (i, 0)),
            full((1, OUT)),
        ],
        out_specs=pl.BlockSpec((R, OUT), lambda i: (i, 0)),
        out_shape=jax.ShapeDtypeStruct((N, OUT), jnp.float32),
    )(aggp, aux, b_gcn.reshape(1, OUT))
    return out


# speedup vs baseline: 25.2198x; 1.4265x over previous
import functools

import jax
import jax.numpy as jnp
from jax import lax
from jax.experimental import pallas as pl
from jax.experimental.pallas import tpu as pltpu
from jax.experimental.pallas import tpu_sc as plsc

N = 10000
E = 320000
F = 128
HID = 128
OUT = 2
EGO = HID // 2

NC = 2
NS = 16
NW = NC * NS
CH = 128
CPT = 80
E_PAD = NW * CPT * CH
ROWS_PER_SUB = 632
N_PAD = NS * ROWS_PER_SUB

_mesh = plsc.VectorSubcoreMesh(core_axis_name="c", subcore_axis_name="s")


def _ids():
    cid = lax.axis_index("c")
    sid = lax.axis_index("s")
    wid = sid * NC + cid
    return cid, sid, wid


FH = F // 2


@functools.partial(
    pl.kernel,
    out_type=(
        jax.ShapeDtypeStruct((NC, N_PAD, FH), jnp.float32),
        jax.ShapeDtypeStruct((NC, N_PAD, FH), jnp.float32),
        jax.ShapeDtypeStruct((NC, N_PAD, 8), jnp.float32),
        jax.ShapeDtypeStruct((NC, N_PAD, 8), jnp.float32),
    ),
    mesh=_mesh,
    scratch_types=[
        pltpu.VMEM((CPT, CH), jnp.int32),
        pltpu.VMEM((CPT, CH), jnp.int32),
        pltpu.VMEM((CH, FH), jnp.float32),
        pltpu.VMEM((CH, FH), jnp.float32),
        pltpu.VMEM((CH, 8), jnp.float32),
        pltpu.SemaphoreType.DMA,
        pltpu.SemaphoreType.DMA,
        pltpu.VMEM_SHARED((N_PAD, FH), jnp.float32),
        pltpu.VMEM_SHARED((N_PAD, 8), jnp.float32),
    ],
    compiler_params=pltpu.CompilerParams(use_tc_tiling_on_sc=False),
)
def _agg_kernel(xlo, xhi, src3d, dst3d, zeros_f, zeros_d, ones_hbm,
                acclo_out, acchi_out, degs_out, degd_out,
                src_b, dst_b, rows0, rows1, ones_b, sg0, sg1, acc_sh, deg_sh):
    cid, sid, wid = _ids()
    row0 = sid * ROWS_PER_SUB
    sl = pl.ds(row0, ROWS_PER_SUB)
    pltpu.sync_copy(zeros_f, acc_sh.at[sl])
    pltpu.sync_copy(zeros_d, deg_sh.at[sl])
    pltpu.sync_copy(src3d.at[wid], src_b)
    pltpu.sync_copy(dst3d.at[wid], dst_b)
    pltpu.sync_copy(ones_hbm, ones_b)
    plsc.subcore_barrier()

    def run_phase(xh, deg_by_src):
        pltpu.async_copy(xh.at[dst_b.at[0]], rows0, sg0)
        pltpu.async_copy(xh.at[dst_b.at[1]], rows1, sg1)

        def body(i, carry):
            for par, (rb, sem) in enumerate(((rows0, sg0), (rows1, sg1))):
                k = 2 * i + par
                pltpu.make_async_copy(xh.at[dst_b.at[k]], rb, sem).wait()
                pltpu.sync_copy(rb, acc_sh.at[src_b.at[k]], add=True)
                deg_row = src_b.at[k] if deg_by_src else dst_b.at[k]
                pltpu.sync_copy(ones_b, deg_sh.at[deg_row], add=True)
                kn = jnp.minimum(k + 2, CPT - 1)
                pltpu.async_copy(xh.at[dst_b.at[kn]], rb, sem)
            return carry

        lax.fori_loop(0, CPT // 2, body, 0)
        pltpu.make_async_copy(xh.at[dst_b.at[0]], rows0, sg0).wait()
        pltpu.make_async_copy(xh.at[dst_b.at[0]], rows1, sg1).wait()

    run_phase(xlo, True)
    plsc.subcore_barrier()
    pltpu.sync_copy(acc_sh.at[sl], acclo_out.at[cid, sl])
    pltpu.sync_copy(deg_sh.at[sl], degs_out.at[cid, sl])
    pltpu.sync_copy(zeros_f, acc_sh.at[sl])
    pltpu.sync_copy(zeros_d, deg_sh.at[sl])
    plsc.subcore_barrier()
    run_phase(xhi, False)
    plsc.subcore_barrier()
    pltpu.sync_copy(acc_sh.at[sl], acchi_out.at[cid, sl])
    pltpu.sync_copy(deg_sh.at[sl], degd_out.at[cid, sl])


@functools.partial(
    pl.kernel,
    out_type=jax.ShapeDtypeStruct((NC, N_PAD, 16), jnp.float32),
    mesh=_mesh,
    scratch_types=[
        pltpu.VMEM((CPT, CH), jnp.int32),
        pltpu.VMEM((CPT, CH), jnp.int32),
        pltpu.VMEM((CH, 16), jnp.float32),
        pltpu.VMEM((CH, 16), jnp.float32),
        pltpu.SemaphoreType.DMA,
        pltpu.SemaphoreType.DMA,
        pltpu.VMEM_SHARED((N_PAD, 16), jnp.float32),
    ],
    compiler_params=pltpu.CompilerParams(use_tc_tiling_on_sc=False),
)
def _gcn_agg_kernel(z_hbm, src3d, dst3d, zeros_d, agg_out,
                    src_b, dst_b, rows0, rows1, sg0, sg1, agg_sh):
    cid, sid, wid = _ids()
    row0 = sid * ROWS_PER_SUB
    pltpu.sync_copy(zeros_d, agg_sh.at[pl.ds(row0, ROWS_PER_SUB)])
    pltpu.sync_copy(src3d.at[wid], src_b)
    pltpu.sync_copy(dst3d.at[wid], dst_b)
    plsc.subcore_barrier()

    pltpu.async_copy(z_hbm.at[src_b.at[0]], rows0, sg0)
    pltpu.async_copy(z_hbm.at[src_b.at[1]], rows1, sg1)

    def body(i, carry):
        for par, (rb, sem) in enumerate(((rows0, sg0), (rows1, sg1))):
            k = 2 * i + par
            pltpu.make_async_copy(z_hbm.at[src_b.at[k]], rb, sem).wait()
            pltpu.sync_copy(rb, agg_sh.at[dst_b.at[k]], add=True)
            kn = jnp.minimum(k + 2, CPT - 1)
            pltpu.async_copy(z_hbm.at[src_b.at[kn]], rb, sem)
        return carry

    lax.fori_loop(0, CPT // 2, body, 0)
    pltpu.make_async_copy(z_hbm.at[src_b.at[0]], rows0, sg0).wait()
    pltpu.make_async_copy(z_hbm.at[src_b.at[0]], rows1, sg1).wait()
    plsc.subcore_barrier()
    sl = pl.ds(row0, ROWS_PER_SUB)
    pltpu.sync_copy(agg_sh.at[sl], agg_out.at[cid, sl])


def _dense_body(x_ref, acclo_ref, acchi_ref, degsp_ref, degdp_ref, fi_ref,
                wm_ref, bm_ref, we_ref, be_ref, wn_ref, bnb_ref,
                gw_ref, gb_ref, gam_ref, bet_ref, wg_ref,
                z_ref, aux_ref):
    sig_fi = jax.nn.sigmoid(fi_ref[...])
    xs = x_ref[...] * sig_fi
    nbr = jnp.concatenate(
        [acclo_ref[0] + acclo_ref[1], acchi_ref[0] + acchi_ref[1]], axis=1) * sig_fi
    deg = degsp_ref[0, :, 0:1] + degsp_ref[1, :, 0:1]
    mean = nbr / jnp.maximum(deg, 1.0)
    xn = xs / jnp.maximum(jnp.sqrt(jnp.sum(xs * xs, axis=1, keepdims=True)), 1e-12)
    mn = mean / jnp.maximum(jnp.sqrt(jnp.sum(mean * mean, axis=1, keepdims=True)), 1e-12)
    sim = jnp.sum(xn * mn, axis=1, keepdims=True)
    sim = jnp.where(deg > 0, sim, 1.0)
    delta = jax.nn.sigmoid(deg * (1.0 - sim) / 10.0 - 0.5)

    dot = lambda a, w: lax.dot_general(a, w, (((1,), (1,)), ((), ())),
                                       preferred_element_type=jnp.float32)
    h_mean = dot((xs + mean) * 0.5, wm_ref[...]) + bm_ref[...]
    h_ego = dot(xs, we_ref[...]) + be_ref[...]
    h_nbr = dot(mean, wn_ref[...]) + bnb_ref[...]
    h_cat = jnp.concatenate([h_ego, h_nbr], axis=1)
    gate = jax.nn.sigmoid(gw_ref[0, 0] * delta + gb_ref[0, 0])
    h = (1.0 - gate) * h_mean + gate * h_cat
    h = h * (1.0 / jnp.sqrt(1.0 + 1e-5)) * gam_ref[...] + bet_ref[...]
    h = jnp.maximum(h, 0.0)
    hw = dot(h, wg_ref[...])

    degd = degdp_ref[0, :, 0:1] + degdp_ref[1, :, 0:1]
    dinv = 1.0 / jnp.sqrt(degd + 1.0)
    z = dinv * hw
    r = z.shape[0]
    z_ref[...] = jnp.concatenate([z, jnp.zeros((r, 16 - OUT), jnp.float32)], axis=1)
    aux_ref[...] = jnp.concatenate(
        [dinv, jnp.zeros((r, 1), jnp.float32), dinv * z,
         jnp.zeros((r, 12), jnp.float32)], axis=1)


def _final_body(aggp_ref, aux_ref, bg_ref, out_ref):
    agg = aggp_ref[0] + aggp_ref[1]
    out_ref[...] = aux_ref[:, 0:1] * agg[:, 0:OUT] + aux_ref[:, 2:2 + OUT] + bg_ref[...]


def kernel(x, edge_index, feature_importance, W_mean, b_mean, W_ego, b_ego,
           W_nbr, b_nbr, gate_w, gate_b, bn_gamma, bn_beta, W_gcn, b_gcn):
    dummy = N + (jnp.arange(E_PAD - E, dtype=jnp.int32) % (N_PAD - N))
    src3d = jnp.concatenate([edge_index[0], dummy]).reshape(NW, CPT, CH)
    dst3d = jnp.concatenate([edge_index[1], dummy]).reshape(NW, CPT, CH)
    xpad = jnp.concatenate([x, jnp.zeros((N_PAD - N, F), jnp.float32)])
    zeros_f = jnp.zeros((ROWS_PER_SUB, FH), jnp.float32)
    zeros_8 = jnp.zeros((ROWS_PER_SUB, 8), jnp.float32)
    zeros_d = jnp.zeros((ROWS_PER_SUB, 16), jnp.float32)
    ones8 = jnp.ones((CH, 8), jnp.float32)

    acclo, acchi, degsp, degdp = _agg_kernel(
        xpad[:, :FH], xpad[:, FH:], src3d, dst3d, zeros_f, zeros_8, ones8)

    R = 1000
    grid = N // R
    full = lambda shape: pl.BlockSpec(shape, lambda i: (0,) * len(shape))
    zpad, aux = pl.pallas_call(
        _dense_body,
        grid=(grid,),
        in_specs=[
            pl.BlockSpec((R, F), lambda i: (i, 0)),
            pl.BlockSpec((NC, R, FH), lambda i: (0, i, 0)),
            pl.BlockSpec((NC, R, FH), lambda i: (0, i, 0)),
            pl.BlockSpec((NC, R, 8), lambda i: (0, i, 0)),
            pl.BlockSpec((NC, R, 8), lambda i: (0, i, 0)),
            full((1, F)),
            full((HID, F)), full((1, HID)),
            full((EGO, F)), full((1, EGO)),
            full((HID - EGO, F)), full((1, HID - EGO)),
            full((1, 1)), full((1, 1)),
            full((1, HID)), full((1, HID)),
            full((OUT, HID)),
        ],
        out_specs=[
            pl.BlockSpec((R, 16), lambda i: (i, 0)),
            pl.BlockSpec((R, 16), lambda i: (i, 0)),
        ],
        out_shape=[
            jax.ShapeDtypeStruct((N_PAD, 16), jnp.float32),
            jax.ShapeDtypeStruct((N, 16), jnp.float32),
        ],
    )(x, acclo, acchi, degsp, degdp,
      feature_importance.reshape(1, F),
      W_mean, b_mean.reshape(1, HID),
      W_ego, b_ego.reshape(1, EGO),
      W_nbr, b_nbr.reshape(1, HID - EGO),
      gate_w.reshape(1, 1), gate_b.reshape(1, 1),
      bn_gamma.reshape(1, HID), bn_beta.reshape(1, HID),
      W_gcn)

    aggp = _gcn_agg_kernel(zpad, src3d, dst3d, zeros_d)

    out = pl.pallas_call(
        _final_body,
        grid=(grid,),
        in_specs=[
            pl.BlockSpec((NC, R, 16), lambda i: (0, i, 0)),
            pl.BlockSpec((R, 16), lambda i: (i, 0)),
            full((1, OUT)),
        ],
        out_specs=pl.BlockSpec((R, OUT), lambda i: (i, 0)),
        out_shape=jax.ShapeDtypeStruct((N, OUT), jnp.float32),
    )(aggp, aux, b_gcn.reshape(1, OUT))
    return out


# speedup vs baseline: 29.3959x; 1.1656x over previous
import functools

import jax
import jax.numpy as jnp
from jax import lax
from jax.experimental import pallas as pl
from jax.experimental.pallas import tpu as pltpu
from jax.experimental.pallas import tpu_sc as plsc

N = 10000
E = 320000
F = 128
HID = 128
OUT = 2
EGO = HID // 2

NC = 2
NS = 16
NW = NC * NS
CH = 128
CPT = 80
E_PAD = NW * CPT * CH
ROWS_PER_SUB = 632
N_PAD = NS * ROWS_PER_SUB

_mesh = plsc.VectorSubcoreMesh(core_axis_name="c", subcore_axis_name="s")


def _ids():
    cid = lax.axis_index("c")
    sid = lax.axis_index("s")
    wid = sid * NC + cid
    return cid, sid, wid


FH = F // 2


@functools.partial(
    pl.kernel,
    out_type=(
        jax.ShapeDtypeStruct((NC, N_PAD, FH), jnp.float32),
        jax.ShapeDtypeStruct((NC, N_PAD, FH), jnp.float32),
        jax.ShapeDtypeStruct((NC, N_PAD, 8), jnp.float32),
        jax.ShapeDtypeStruct((NC, N_PAD, 8), jnp.float32),
    ),
    mesh=_mesh,
    scratch_types=[
        pltpu.VMEM((CPT, CH), jnp.int32),
        pltpu.VMEM((CPT, CH), jnp.int32),
        pltpu.VMEM((CH, FH), jnp.float32),
        pltpu.VMEM((CH, FH), jnp.float32),
        pltpu.VMEM((CH, FH), jnp.float32),
        pltpu.VMEM((CH, FH), jnp.float32),
        pltpu.VMEM((CH, 8), jnp.float32),
        pltpu.SemaphoreType.DMA,
        pltpu.SemaphoreType.DMA,
        pltpu.SemaphoreType.DMA,
        pltpu.SemaphoreType.DMA,
        pltpu.SemaphoreType.DMA,
        pltpu.VMEM_SHARED((N_PAD, FH), jnp.float32),
        pltpu.VMEM_SHARED((N_PAD, 8), jnp.float32),
    ],
    compiler_params=pltpu.CompilerParams(use_tc_tiling_on_sc=False),
)
def _agg_kernel(xlo, xhi, src3d, dst3d, zeros_f, zeros_d, ones_hbm,
                acclo_out, acchi_out, degs_out, degd_out,
                src_b, dst_b, rows0, rows1, rows2, rows3, ones_b,
                sg0, sg1, sg2, sg3, so, acc_sh, deg_sh):
    cid, sid, wid = _ids()
    row0 = sid * ROWS_PER_SUB
    sl = pl.ds(row0, ROWS_PER_SUB)
    pltpu.sync_copy(zeros_f, acc_sh.at[sl])
    pltpu.sync_copy(zeros_d, deg_sh.at[sl])
    pltpu.sync_copy(src3d.at[wid], src_b)
    pltpu.sync_copy(dst3d.at[wid], dst_b)
    pltpu.sync_copy(ones_hbm, ones_b)
    plsc.subcore_barrier()

    ring = ((rows0, sg0), (rows1, sg1), (rows2, sg2), (rows3, sg3))

    def run_phase(xh, deg_by_src):
        for b, (rb, sem) in enumerate(ring):
            pltpu.async_copy(xh.at[dst_b.at[b]], rb, sem)

        def body(i, carry):
            for par, (rb, sem) in enumerate(ring):
                k = 4 * i + par
                pltpu.make_async_copy(xh.at[dst_b.at[k]], rb, sem).wait()
                pltpu.sync_copy(rb, acc_sh.at[src_b.at[k]], add=True)
                deg_row = src_b.at[k] if deg_by_src else dst_b.at[k]
                pltpu.async_copy(ones_b, deg_sh.at[deg_row], so, add=True)
                kn = jnp.minimum(k + 4, CPT - 1)
                pltpu.async_copy(xh.at[dst_b.at[kn]], rb, sem)
            return carry

        lax.fori_loop(0, CPT // 4, body, 0)
        for rb, sem in ring:
            pltpu.make_async_copy(xh.at[dst_b.at[0]], rb, sem).wait()

        def drain(k, carry):
            pltpu.make_async_copy(ones_b, deg_sh.at[src_b.at[0]], so).wait()
            return carry

        lax.fori_loop(0, CPT, drain, 0)

    run_phase(xlo, True)
    plsc.subcore_barrier()
    pltpu.sync_copy(acc_sh.at[sl], acclo_out.at[cid, sl])
    pltpu.sync_copy(deg_sh.at[sl], degs_out.at[cid, sl])
    pltpu.sync_copy(zeros_f, acc_sh.at[sl])
    pltpu.sync_copy(zeros_d, deg_sh.at[sl])
    plsc.subcore_barrier()
    run_phase(xhi, False)
    plsc.subcore_barrier()
    pltpu.sync_copy(acc_sh.at[sl], acchi_out.at[cid, sl])
    pltpu.sync_copy(deg_sh.at[sl], degd_out.at[cid, sl])


@functools.partial(
    pl.kernel,
    out_type=jax.ShapeDtypeStruct((NC, N_PAD, 16), jnp.float32),
    mesh=_mesh,
    scratch_types=[
        pltpu.VMEM((CPT, CH), jnp.int32),
        pltpu.VMEM((CPT, CH), jnp.int32),
        pltpu.VMEM((CH, 16), jnp.float32),
        pltpu.VMEM((CH, 16), jnp.float32),
        pltpu.VMEM((CH, 16), jnp.float32),
        pltpu.VMEM((CH, 16), jnp.float32),
        pltpu.SemaphoreType.DMA,
        pltpu.SemaphoreType.DMA,
        pltpu.SemaphoreType.DMA,
        pltpu.SemaphoreType.DMA,
        pltpu.VMEM_SHARED((N_PAD, 16), jnp.float32),
    ],
    compiler_params=pltpu.CompilerParams(use_tc_tiling_on_sc=False),
)
def _gcn_agg_kernel(z_hbm, src3d, dst3d, zeros_d, agg_out,
                    src_b, dst_b, rows0, rows1, rows2, rows3,
                    sg0, sg1, sg2, sg3, agg_sh):
    cid, sid, wid = _ids()
    row0 = sid * ROWS_PER_SUB
    pltpu.sync_copy(zeros_d, agg_sh.at[pl.ds(row0, ROWS_PER_SUB)])
    pltpu.sync_copy(src3d.at[wid], src_b)
    pltpu.sync_copy(dst3d.at[wid], dst_b)
    plsc.subcore_barrier()

    ring = ((rows0, sg0), (rows1, sg1), (rows2, sg2), (rows3, sg3))
    for b, (rb, sem) in enumerate(ring):
        pltpu.async_copy(z_hbm.at[src_b.at[b]], rb, sem)

    def body(i, carry):
        for par, (rb, sem) in enumerate(ring):
            k = 4 * i + par
            pltpu.make_async_copy(z_hbm.at[src_b.at[k]], rb, sem).wait()
            pltpu.sync_copy(rb, agg_sh.at[dst_b.at[k]], add=True)
            kn = jnp.minimum(k + 4, CPT - 1)
            pltpu.async_copy(z_hbm.at[src_b.at[kn]], rb, sem)
        return carry

    lax.fori_loop(0, CPT // 4, body, 0)
    for rb, sem in ring:
        pltpu.make_async_copy(z_hbm.at[src_b.at[0]], rb, sem).wait()
    plsc.subcore_barrier()
    sl = pl.ds(row0, ROWS_PER_SUB)
    pltpu.sync_copy(agg_sh.at[sl], agg_out.at[cid, sl])


def _dense_body(x_ref, acclo_ref, acchi_ref, degsp_ref, degdp_ref, fi_ref,
                wm_ref, bm_ref, we_ref, be_ref, wn_ref, bnb_ref,
                gw_ref, gb_ref, gam_ref, bet_ref, wg_ref,
                z_ref, aux_ref):
    sig_fi = jax.nn.sigmoid(fi_ref[...])
    xs = x_ref[...] * sig_fi
    nbr = jnp.concatenate(
        [acclo_ref[0] + acclo_ref[1], acchi_ref[0] + acchi_ref[1]], axis=1) * sig_fi
    deg = degsp_ref[0, :, 0:1] + degsp_ref[1, :, 0:1]
    mean = nbr / jnp.maximum(deg, 1.0)
    xn = xs / jnp.maximum(jnp.sqrt(jnp.sum(xs * xs, axis=1, keepdims=True)), 1e-12)
    mn = mean / jnp.maximum(jnp.sqrt(jnp.sum(mean * mean, axis=1, keepdims=True)), 1e-12)
    sim = jnp.sum(xn * mn, axis=1, keepdims=True)
    sim = jnp.where(deg > 0, sim, 1.0)
    delta = jax.nn.sigmoid(deg * (1.0 - sim) / 10.0 - 0.5)

    dot = lambda a, w: lax.dot_general(a, w, (((1,), (1,)), ((), ())),
                                       preferred_element_type=jnp.float32)
    h_mean = dot((xs + mean) * 0.5, wm_ref[...]) + bm_ref[...]
    h_ego = dot(xs, we_ref[...]) + be_ref[...]
    h_nbr = dot(mean, wn_ref[...]) + bnb_ref[...]
    h_cat = jnp.concatenate([h_ego, h_nbr], axis=1)
    gate = jax.nn.sigmoid(gw_ref[0, 0] * delta + gb_ref[0, 0])
    h = (1.0 - gate) * h_mean + gate * h_cat
    h = h * (1.0 / jnp.sqrt(1.0 + 1e-5)) * gam_ref[...] + bet_ref[...]
    h = jnp.maximum(h, 0.0)
    hw = dot(h, wg_ref[...])

    degd = degdp_ref[0, :, 0:1] + degdp_ref[1, :, 0:1]
    dinv = 1.0 / jnp.sqrt(degd + 1.0)
    z = dinv * hw
    r = z.shape[0]
    z_ref[...] = jnp.concatenate([z, jnp.zeros((r, 16 - OUT), jnp.float32)], axis=1)
    aux_ref[...] = jnp.concatenate(
        [dinv, jnp.zeros((r, 1), jnp.float32), dinv * z,
         jnp.zeros((r, 12), jnp.float32)], axis=1)


def _final_body(aggp_ref, aux_ref, bg_ref, out_ref):
    agg = aggp_ref[0] + aggp_ref[1]
    out_ref[...] = aux_ref[:, 0:1] * agg[:, 0:OUT] + aux_ref[:, 2:2 + OUT] + bg_ref[...]


def kernel(x, edge_index, feature_importance, W_mean, b_mean, W_ego, b_ego,
           W_nbr, b_nbr, gate_w, gate_b, bn_gamma, bn_beta, W_gcn, b_gcn):
    dummy = N + (jnp.arange(E_PAD - E, dtype=jnp.int32) % (N_PAD - N))
    src3d = jnp.concatenate([edge_index[0], dummy]).reshape(NW, CPT, CH)
    dst3d = jnp.concatenate([edge_index[1], dummy]).reshape(NW, CPT, CH)
    xpad = jnp.concatenate([x, jnp.zeros((N_PAD - N, F), jnp.float32)])
    zeros_f = jnp.zeros((ROWS_PER_SUB, FH), jnp.float32)
    zeros_8 = jnp.zeros((ROWS_PER_SUB, 8), jnp.float32)
    zeros_d = jnp.zeros((ROWS_PER_SUB, 16), jnp.float32)
    ones8 = jnp.ones((CH, 8), jnp.float32)

    acclo, acchi, degsp, degdp = _agg_kernel(
        xpad[:, :FH], xpad[:, FH:], src3d, dst3d, zeros_f, zeros_8, ones8)

    R = 1000
    grid = N // R
    full = lambda shape: pl.BlockSpec(shape, lambda i: (0,) * len(shape))
    zpad, aux = pl.pallas_call(
        _dense_body,
        grid=(grid,),
        in_specs=[
            pl.BlockSpec((R, F), lambda i: (i, 0)),
            pl.BlockSpec((NC, R, FH), lambda i: (0, i, 0)),
            pl.BlockSpec((NC, R, FH), lambda i: (0, i, 0)),
            pl.BlockSpec((NC, R, 8), lambda i: (0, i, 0)),
            pl.BlockSpec((NC, R, 8), lambda i: (0, i, 0)),
            full((1, F)),
            full((HID, F)), full((1, HID)),
            full((EGO, F)), full((1, EGO)),
            full((HID - EGO, F)), full((1, HID - EGO)),
            full((1, 1)), full((1, 1)),
            full((1, HID)), full((1, HID)),
            full((OUT, HID)),
        ],
        out_specs=[
            pl.BlockSpec((R, 16), lambda i: (i, 0)),
            pl.BlockSpec((R, 16), lambda i: (i, 0)),
        ],
        out_shape=[
            jax.ShapeDtypeStruct((N_PAD, 16), jnp.float32),
            jax.ShapeDtypeStruct((N, 16), jnp.float32),
        ],
    )(x, acclo, acchi, degsp, degdp,
      feature_importance.reshape(1, F),
      W_mean, b_mean.reshape(1, HID),
      W_ego, b_ego.reshape(1, EGO),
      W_nbr, b_nbr.reshape(1, HID - EGO),
      gate_w.reshape(1, 1), gate_b.reshape(1, 1),
      bn_gamma.reshape(1, HID), bn_beta.reshape(1, HID),
      W_gcn)

    aggp = _gcn_agg_kernel(zpad, src3d, dst3d, zeros_d)

    out = pl.pallas_call(
        _final_body,
        grid=(grid,),
        in_specs=[
            pl.BlockSpec((NC, R, 16), lambda i: (0, i, 0)),
            pl.BlockSpec((R, 16), lambda i: (i, 0)),
            full((1, OUT)),
        ],
        out_specs=pl.BlockSpec((R, OUT), lambda i: (i, 0)),
        out_shape=jax.ShapeDtypeStruct((N, OUT), jnp.float32),
    )(aggp, aux, b_gcn.reshape(1, OUT))
    return out
